# Initial kernel scaffold; baseline (speedup 1.0000x reference)
import jax
import jax.numpy as jnp
from jax.experimental import pallas as pl


def kernel(node_state, adjacency, point_enc, relation_enc, point_enc_w, relation_enc_w, q_w, k_w, v_w, fc_w, fc_b):
    raise NotImplementedError("write your pallas kernel here")


# speedup vs baseline: 19.9972x; 19.9972x over previous
import functools

import jax
import jax.numpy as jnp
from jax import lax
from jax.experimental import pallas as pl
from jax.experimental.pallas import tpu as pltpu
from jax.experimental.pallas import tpu_sc as plsc

N = 10000
DEG = 16
U = 256
RELS = 8
ENTS = 4
EDGES = N * DEG

NBLK = 1000
SBLK = 200
NEG = -1000000000.0

def _proj_body(ns_ref, pe_ref, adj_ref, rel_ref, pw_ref, qw_ref, kw_ref,
               vw_ref, relw_ref, nso_ref, qso_ref, lin_ref, kwo_ref, vwo_ref):
    x = ns_ref[...]
    pe = pe_ref[...]
    acc = jnp.zeros((NBLK, U), jnp.float32)
    for e in range(ENTS):
        pm = jnp.where(pe == e, 1.0, 0.0)
        acc = acc + pm * jnp.dot(x, pw_ref[e],
                                 preferred_element_type=jnp.float32)
    nso_ref[...] = acc
    qso_ref[...] = jnp.dot(acc, qw_ref[...],
                           preferred_element_type=jnp.float32) * (1.0 / U)
    adj = adj_ref[...]
    rel = rel_ref[...]
    lin_ref[...] = rel * N + jnp.maximum(adj - 1, 0)

    @pl.when(pl.program_id(0) == 0)
    def _():
        for r in range(RELS):
            kwo_ref[r] = jnp.dot(relw_ref[r], kw_ref[...],
                                 preferred_element_type=jnp.float32)
            vwo_ref[r] = jnp.dot(relw_ref[r], vw_ref[...],
                                 preferred_element_type=jnp.float32)


def _project(node_state, point_enc, adjacency, relation_enc, pw, qw, kw, vw,
             relw):
    grid = (N // NBLK,)
    return pl.pallas_call(
        _proj_body,
        grid=grid,
        in_specs=[
            pl.BlockSpec((NBLK, U), lambda i: (i, 0)),
            pl.BlockSpec((NBLK, 1), lambda i: (i, 0)),
            pl.BlockSpec((NBLK, DEG), lambda i: (i, 0)),
            pl.BlockSpec((NBLK, DEG), lambda i: (i, 0)),
            pl.BlockSpec((ENTS, U, U), lambda i: (0, 0, 0)),
            pl.BlockSpec((U, U), lambda i: (0, 0)),
            pl.BlockSpec((U, U), lambda i: (0, 0)),
            pl.BlockSpec((U, U), lambda i: (0, 0)),
            pl.BlockSpec((RELS, U, U), lambda i: (0, 0, 0)),
        ],
        out_specs=[
            pl.BlockSpec((NBLK, U), lambda i: (i, 0)),
            pl.BlockSpec((NBLK, U), lambda i: (i, 0)),
            pl.BlockSpec((NBLK, DEG), lambda i: (i, 0)),
            pl.BlockSpec((RELS, U, U), lambda i: (0, 0, 0)),
            pl.BlockSpec((RELS, U, U), lambda i: (0, 0, 0)),
        ],
        out_shape=[
            jax.ShapeDtypeStruct((N, U), jnp.float32),
            jax.ShapeDtypeStruct((N, U), jnp.float32),
            jax.ShapeDtypeStruct((N, DEG), jnp.int32),
            jax.ShapeDtypeStruct((RELS, U, U), jnp.float32),
            jax.ShapeDtypeStruct((RELS, U, U), jnp.float32),
        ],
    )(node_state, point_enc, adjacency, relation_enc, pw, qw, kw, vw, relw)


def _tables_body(ns_ref, kwf_ref, vwf_ref, kt_ref, vt_ref):
    x = ns_ref[...]
    kt_ref[0] = jnp.dot(x, kwf_ref[0], preferred_element_type=jnp.float32)
    vt_ref[0] = jnp.dot(x, vwf_ref[0], preferred_element_type=jnp.float32)


def _tables(ns, kwf, vwf):
    grid = (RELS, N // NBLK)
    return pl.pallas_call(
        _tables_body,
        grid=grid,
        in_specs=[
            pl.BlockSpec((NBLK, U), lambda r, i: (i, 0)),
            pl.BlockSpec((1, U, U), lambda r, i: (r, 0, 0)),
            pl.BlockSpec((1, U, U), lambda r, i: (r, 0, 0)),
        ],
        out_specs=[
            pl.BlockSpec((1, NBLK, U), lambda r, i: (r, i, 0)),
            pl.BlockSpec((1, NBLK, U), lambda r, i: (r, i, 0)),
        ],
        out_shape=[
            jax.ShapeDtypeStruct((RELS, N, U), jnp.float32),
            jax.ShapeDtypeStruct((RELS, N, U), jnp.float32),
        ],
    )(ns, kwf, vwf)


_SC_CHUNK = 40


def _sc_gather(lin_flat, kt_flat, vt_flat):
    info = plsc.get_sparse_core_info()
    nw = info.num_cores * info.num_subcores
    per_w = EDGES // nw
    nchunk = per_w // _SC_CHUNK
    mesh = plsc.VectorSubcoreMesh(core_axis_name="c", subcore_axis_name="s")

    @functools.partial(
        pl.kernel,
        mesh=mesh,
        out_type=(jax.ShapeDtypeStruct((EDGES, U), jnp.float32),
                  jax.ShapeDtypeStruct((EDGES, U), jnp.float32)),
        scratch_types=[
            pltpu.VMEM((_SC_CHUNK,), jnp.int32),
            pltpu.VMEM((_SC_CHUNK, U), jnp.float32),
            pltpu.VMEM((_SC_CHUNK, U), jnp.float32),
            pltpu.SemaphoreType.DMA,
            pltpu.SemaphoreType.DMA,
        ],
    )
    def gather(lin_hbm, kt_hbm, vt_hbm, kg_hbm, vg_hbm, idx_v, kbuf, vbuf,
               semk, semv):
        wid = lax.axis_index("s") * info.num_cores + lax.axis_index("c")
        base = wid * per_w

        def chunk(i, carry):
            off = base + i * _SC_CHUNK
            pltpu.sync_copy(lin_hbm.at[pl.ds(off, _SC_CHUNK)], idx_v)
            ck = pltpu.async_copy(kt_hbm.at[idx_v], kbuf, semk)
            cv = pltpu.async_copy(vt_hbm.at[idx_v], vbuf, semv)
            ck.wait()
            cv.wait()
            pltpu.sync_copy(kbuf, kg_hbm.at[pl.ds(off, _SC_CHUNK)])
            pltpu.sync_copy(vbuf, vg_hbm.at[pl.ds(off, _SC_CHUNK)])
            return carry

        lax.fori_loop(0, nchunk, chunk, 0)

    return gather(lin_flat, kt_flat, vt_flat)


def _scores_body(qs_ref, kg_ref, rel_ref, adj_ref, raw_ref):
    qs = qs_ref[...]
    dots = jnp.sum(kg_ref[...] * qs[:, None, :], axis=-1)
    rel = rel_ref[...]
    adj = adj_ref[...]
    raw = jnp.where(adj == 0, 0.0, dots)
    raw_ref[...] = jnp.where(rel == 0, NEG, raw)


def _scores(qs, kg, relation_enc, adjacency):
    grid = (N // SBLK,)
    return pl.pallas_call(
        _scores_body,
        grid=grid,
        in_specs=[
            pl.BlockSpec((SBLK, U), lambda i: (i, 0)),
            pl.BlockSpec((SBLK, DEG, U), lambda i: (i, 0, 0)),
            pl.BlockSpec((SBLK, DEG), lambda i: (i, 0)),
            pl.BlockSpec((SBLK, DEG), lambda i: (i, 0)),
        ],
        out_specs=pl.BlockSpec((SBLK, DEG), lambda i: (i, 0)),
        out_shape=jax.ShapeDtypeStruct((N, DEG), jnp.float32),
    )(qs, kg, relation_enc, adjacency)


def _softmax_body(raw_ref, w_ref):
    raw = raw_ref[...]
    m = jnp.max(raw, axis=0, keepdims=True)
    e = jnp.exp(raw - m)
    s = jnp.sum(e, axis=0, keepdims=True)
    w_ref[...] = e / s


def _softmax0(raw):
    return pl.pallas_call(
        _softmax_body,
        out_shape=jax.ShapeDtypeStruct((N, DEG), jnp.float32),
    )(raw)


def _agg_body(w_ref, adj_ref, vg_ref, ns_ref, fcw_ref, fcb_ref, out_ref):
    w = jnp.where(adj_ref[...] == 0, 0.0, w_ref[...])
    agg = jnp.sum(w[:, :, None] * vg_ref[...], axis=1)
    fc = lax.dot_general(agg, fcw_ref[...], (((1,), (1,)), ((), ())),
                         preferred_element_type=jnp.float32) + fcb_ref[...]
    out_ref[...] = ns_ref[...] + jnp.maximum(fc, 0.0)


def _aggregate(w, adjacency, vg, ns, fc_w, fc_b):
    grid = (N // SBLK,)
    return pl.pallas_call(
        _agg_body,
        grid=grid,
        in_specs=[
            pl.BlockSpec((SBLK, DEG), lambda i: (i, 0)),
            pl.BlockSpec((SBLK, DEG), lambda i: (i, 0)),
            pl.BlockSpec((SBLK, DEG, U), lambda i: (i, 0, 0)),
            pl.BlockSpec((SBLK, U), lambda i: (i, 0)),
            pl.BlockSpec((U, U), lambda i: (0, 0)),
            pl.BlockSpec((1, U), lambda i: (0, 0)),
        ],
        out_specs=pl.BlockSpec((SBLK, U), lambda i: (i, 0)),
        out_shape=jax.ShapeDtypeStruct((N, U), jnp.float32),
    )(w, adjacency, vg, ns, fc_w, fc_b)


def kernel(node_state, adjacency, point_enc, relation_enc, point_enc_w,
           relation_enc_w, q_w, k_w, v_w, fc_w, fc_b):
    pe2 = point_enc.reshape(N, 1)
    ns, qs, lin, kwf, vwf = _project(node_state, pe2, adjacency, relation_enc,
                                     point_enc_w, q_w, k_w, v_w,
                                     relation_enc_w)
    kt, vt = _tables(ns, kwf, vwf)
    kg, vg = _sc_gather(lin.reshape(EDGES), kt.reshape(RELS * N, U),
                        vt.reshape(RELS * N, U))
    raw = _scores(qs, kg.reshape(N, DEG, U), relation_enc, adjacency)
    w = _softmax0(raw)
    out = _aggregate(w, adjacency, vg.reshape(N, DEG, U), ns, fc_w,
                     fc_b.reshape(1, U))
    return out
